# Initial kernel scaffold; baseline (speedup 1.0000x reference)
import jax
import jax.numpy as jnp
from jax.experimental import pallas as pl


def kernel(x, proj_W, proj_b, gat1_W, gat2_W, bn1_g, bn1_b, bn1_rm, bn1_rv, bn2_g, bn2_b, bn2_rm, bn2_rv, ln_g, ln_b):
    raise NotImplementedError("write your pallas kernel here")


# speedup vs baseline: 13.3010x; 13.3010x over previous
import functools

import jax
import jax.numpy as jnp
import numpy as np
from jax.experimental import pallas as pl
from jax.experimental.pallas import tpu as pltpu

IN_FEATURES = 20
BN_EPS = 1e-5
LN_EPS = 1e-5

_B = 4096
_BB = 256


def _build_avg_matrix(num_nodes=IN_FEATURES, k=8):
    m = np.zeros((num_nodes, num_nodes), np.float32)
    for i in range(num_nodes):
        for j in range(max(0, i - k // 2), min(num_nodes, i + k // 2 + 1)):
            if i != j:
                m[j, i] = 1.0
    deg = m.sum(axis=1, keepdims=True)
    return m / np.maximum(deg, 1.0)

_M = _build_avg_matrix()


def _elu(v):
    return jnp.where(v > 0, v, jnp.exp(v) - 1.0)


def _body(x_ref, p64_ref, w1_ref, bn1_ref, w2_ref, p128_ref, m_ref, o_ref):
    bb = o_ref.shape[0]
    xcol = x_ref[...]
    h = jnp.maximum(xcol * p64_ref[0:1, :] + p64_ref[1:2, :], 0.0)

    def tail(h1, h2_rows):
        h1 = _elu(h1)
        h1 = h1 * bn1_ref[0:1, :] + bn1_ref[1:2, :]
        h2 = jnp.dot(h1, w2_ref[...], preferred_element_type=jnp.float32)
        return h2

    def finish(h2, rows):
        h2 = _elu(h2)
        h2 = h2 * p128_ref[0:1, :] + p128_ref[1:2, :]
        pooled = h2.reshape(rows, IN_FEATURES, h2.shape[-1]).mean(axis=1)
        mu = pooled.mean(axis=-1, keepdims=True)
        var = jnp.mean((pooled - mu) ** 2, axis=-1, keepdims=True)
        normed = (pooled - mu) * jax.lax.rsqrt(var + LN_EPS)
        return normed * p128_ref[2:3, :] + p128_ref[3:4, :]

    h1 = jnp.dot(h, w1_ref[...], preferred_element_type=jnp.float32)
    h2 = tail(h1, None)
    o_ref[...] = finish(h2, bb)

    @pl.when(pl.program_id(0) == 0)
    def _():
        m = m_ref[...]
        t1 = jnp.dot(m, h1[0:IN_FEATURES, :], preferred_element_type=jnp.float32)
        t2 = tail(t1, None)
        t2 = jnp.dot(m, t2, preferred_element_type=jnp.float32)
        o_ref[0:1, :] = finish(t2, 1)


@functools.partial(jax.jit, static_argnames=())
def kernel(x, proj_W, proj_b, gat1_W, gat2_W, bn1_g, bn1_b, bn1_rm, bn1_rv,
           bn2_g, bn2_b, bn2_rm, bn2_rv, ln_g, ln_b):
    B = x.shape[0]
    bb = _BB if B % _BB == 0 else B
    xcol = x.reshape(-1, 1)

    p64 = jnp.stack([proj_W[:, 0], proj_b])
    w1 = gat1_W.T
    s1 = bn1_g * jax.lax.rsqrt(bn1_rv + BN_EPS)
    bn1 = jnp.stack([s1, bn1_b - bn1_rm * s1])
    w2 = gat2_W.T
    s2 = bn2_g * jax.lax.rsqrt(bn2_rv + BN_EPS)
    p128 = jnp.stack([s2, bn2_b - bn2_rm * s2, ln_g, ln_b])
    m = jnp.asarray(_M)

    nblk = bb * IN_FEATURES
    grid = (B // bb,)
    return pl.pallas_call(
        _body,
        grid=grid,
        in_specs=[
            pl.BlockSpec((nblk, 1), lambda i: (i, 0)),
            pl.BlockSpec(p64.shape, lambda i: (0, 0)),
            pl.BlockSpec(w1.shape, lambda i: (0, 0)),
            pl.BlockSpec(bn1.shape, lambda i: (0, 0)),
            pl.BlockSpec(w2.shape, lambda i: (0, 0)),
            pl.BlockSpec(p128.shape, lambda i: (0, 0)),
            pl.BlockSpec(m.shape, lambda i: (0, 0)),
        ],
        out_specs=pl.BlockSpec((bb, p128.shape[1]), lambda i: (i, 0)),
        out_shape=jax.ShapeDtypeStruct((B, p128.shape[1]), jnp.float32),
        compiler_params=pltpu.CompilerParams(
            dimension_semantics=("arbitrary",),
        ),
    )(xcol, p64, w1, bn1, w2, p128, m)


# speedup vs baseline: 13.8054x; 1.0379x over previous
import jax
import jax.numpy as jnp
import numpy as np
from jax.experimental import pallas as pl
from jax.experimental.pallas import tpu as pltpu

IN_FEATURES = 20
BN_EPS = 1e-5
LN_EPS = 1e-5

_BB = 256


def _build_avg_matrix(num_nodes=IN_FEATURES, k=8):
    m = np.zeros((num_nodes, num_nodes), np.float32)
    for i in range(num_nodes):
        for j in range(max(0, i - k // 2), min(num_nodes, i + k // 2 + 1)):
            if i != j:
                m[j, i] = 1.0
    deg = m.sum(axis=1, keepdims=True)
    return m / np.maximum(deg, 1.0)

_M = _build_avg_matrix()


def _elu(v):
    return jnp.where(v > 0, v, jnp.exp(v) - 1.0)


def _body(x_ref, p64_ref, w1_ref, w2_ref, p128_ref, m_ref, o_ref):
    bb = o_ref.shape[0]
    xcol = x_ref[...]
    hpre = jnp.dot(xcol, p64_ref[0:1, :],
                   preferred_element_type=jnp.float32) + p64_ref[1:2, :]
    h = jnp.maximum(hpre, 0.0)
    h1 = jnp.dot(h, w1_ref[...], preferred_element_type=jnp.float32)

    def second(h1v):
        a1 = _elu(h1v)
        return jnp.dot(a1, w2_ref[...],
                       preferred_element_type=jnp.float32) + p128_ref[0:1, :]

    def finish(h2v, rows):
        a2 = _elu(h2v)
        pooled = a2.reshape(rows, IN_FEATURES, a2.shape[-1]).mean(axis=1)
        pooled = pooled * p128_ref[1:2, :] + p128_ref[2:3, :]
        mu = pooled.mean(axis=-1, keepdims=True)
        var = jnp.mean((pooled - mu) ** 2, axis=-1, keepdims=True)
        normed = (pooled - mu) * jax.lax.rsqrt(var + LN_EPS)
        return normed * p128_ref[3:4, :] + p128_ref[4:5, :]

    o_ref[...] = finish(second(h1), bb)

    @pl.when(pl.program_id(0) == 0)
    def _():
        m = m_ref[...]
        h1c = jnp.dot(m, h1[0:IN_FEATURES, :], preferred_element_type=jnp.float32)
        h2c = jnp.dot(m, second(h1c), preferred_element_type=jnp.float32)
        o_ref[0:1, :] = finish(h2c, 1)


def kernel(x, proj_W, proj_b, gat1_W, gat2_W, bn1_g, bn1_b, bn1_rm, bn1_rv,
           bn2_g, bn2_b, bn2_rm, bn2_rv, ln_g, ln_b):
    B = x.shape[0]
    bb = _BB if B % _BB == 0 else B
    xcol = x.reshape(-1, 1)

    p64 = jnp.stack([proj_W[:, 0], proj_b])
    w1 = gat1_W.T
    s1 = bn1_g * jax.lax.rsqrt(bn1_rv + BN_EPS)
    t1 = bn1_b - bn1_rm * s1
    w2 = gat2_W.T * s1[:, None]
    b2 = t1 @ gat2_W.T
    s2 = bn2_g * jax.lax.rsqrt(bn2_rv + BN_EPS)
    t2 = bn2_b - bn2_rm * s2
    p128 = jnp.stack([b2, s2, t2, ln_g, ln_b])
    m = jnp.asarray(_M)

    nblk = bb * IN_FEATURES
    grid = (B // bb,)
    return pl.pallas_call(
        _body,
        grid=grid,
        in_specs=[
            pl.BlockSpec((nblk, 1), lambda i: (i, 0)),
            pl.BlockSpec(p64.shape, lambda i: (0, 0)),
            pl.BlockSpec(w1.shape, lambda i: (0, 0)),
            pl.BlockSpec(w2.shape, lambda i: (0, 0)),
            pl.BlockSpec(p128.shape, lambda i: (0, 0)),
            pl.BlockSpec(m.shape, lambda i: (0, 0)),
        ],
        out_specs=pl.BlockSpec((bb, p128.shape[1]), lambda i: (i, 0)),
        out_shape=jax.ShapeDtypeStruct((B, p128.shape[1]), jnp.float32),
        compiler_params=pltpu.CompilerParams(
            dimension_semantics=("arbitrary",),
        ),
    )(xcol, p64, w1, w2, p128, m)
